# initial kernel scaffold (unmeasured)
import jax
import jax.numpy as jnp
from jax import lax
from jax.experimental import pallas as pl
from jax.experimental.pallas import tpu as pltpu


def kernel(
    x,
):
    def body(*refs):
        pass

    out_shape = jax.ShapeDtypeStruct(..., jnp.float32)
    return pl.pallas_call(body, out_shape=out_shape)(...)


# baseline (device time: 9366 ns/iter reference)
import jax
import jax.numpy as jnp
from jax import lax
from jax.experimental import pallas as pl
from jax.experimental.pallas import tpu as pltpu

N_DEV = 16


def kernel(x):
    m, n = x.shape

    def body(x_ref, out_ref, total_ref, prefix_ref, send_sem, recv_sem):
        my = lax.axis_index("i")
        left = my - 1
        right = jnp.minimum(my + 1, N_DEV - 1)

        barrier = pltpu.get_barrier_semaphore()

        @pl.when(my > 0)
        def _():
            pl.semaphore_signal(
                barrier, inc=1, device_id=(left,),
                device_id_type=pl.DeviceIdType.MESH,
            )

        @pl.when(my < N_DEV - 1)
        def _():
            pl.semaphore_signal(
                barrier, inc=1, device_id=(right,),
                device_id_type=pl.DeviceIdType.MESH,
            )

        @pl.when((my > 0) & (my < N_DEV - 1))
        def _():
            pl.semaphore_wait(barrier, 2)

        @pl.when((my == 0) | (my == N_DEV - 1))
        def _():
            pl.semaphore_wait(barrier, 1)

        row = lax.broadcasted_iota(jnp.int32, (m, m), 0)
        col = lax.broadcasted_iota(jnp.int32, (m, m), 1)
        tri = (col <= row).astype(jnp.float32)
        out_ref[:, :] = jnp.dot(
            tri, x_ref[:, :], preferred_element_type=jnp.float32
        )
        total_ref[0, :] = out_ref[m - 1, :]

        rdma = pltpu.make_async_remote_copy(
            src_ref=total_ref,
            dst_ref=prefix_ref,
            send_sem=send_sem,
            recv_sem=recv_sem,
            device_id=(right,),
            device_id_type=pl.DeviceIdType.MESH,
        )

        @pl.when(my == 0)
        def _():
            prefix_ref[0, :] = jnp.zeros((n,), jnp.float32)

        @pl.when(my > 0)
        def _():
            rdma.wait_recv()

        total_ref[0, :] = total_ref[0, :] + prefix_ref[0, :]

        @pl.when(my < N_DEV - 1)
        def _():
            rdma.start()
            rdma.wait_send()

        out_ref[:, :] = out_ref[:, :] + prefix_ref[0:1, :]

    return pl.pallas_call(
        body,
        out_shape=jax.ShapeDtypeStruct((m, n), jnp.float32),
        in_specs=[pl.BlockSpec(memory_space=pltpu.VMEM)],
        out_specs=pl.BlockSpec(memory_space=pltpu.VMEM),
        scratch_shapes=[
            pltpu.VMEM((1, n), jnp.float32),
            pltpu.VMEM((1, n), jnp.float32),
            pltpu.SemaphoreType.DMA,
            pltpu.SemaphoreType.DMA,
        ],
        compiler_params=pltpu.CompilerParams(collective_id=0),
    )(x)
